# baseline (speedup 1.0000x reference)
import functools

import jax
import jax.numpy as jnp
from jax import lax
from jax.experimental import pallas as pl
from jax.experimental.pallas import tpu as pltpu
from jax.experimental.pallas import tpu_sc as plsc

_CHUNK = 800


@functools.cache
def _make_gather(B, D, chunk):
    info = plsc.get_sparse_core_info()
    num_workers = info.num_cores * info.num_subcores
    b_per_w = B // num_workers
    n_chunks = b_per_w // chunk
    mesh = plsc.VectorSubcoreMesh(core_axis_name="c", subcore_axis_name="s")

    @functools.partial(
        pl.kernel,
        mesh=mesh,
        out_type=jax.ShapeDtypeStruct((B, 2 * D), jnp.float32),
        scratch_types=[
            pltpu.VMEM((n_chunks, chunk), jnp.int32),
            pltpu.VMEM((chunk, D), jnp.float32),
            pltpu.VMEM((chunk, D), jnp.float32),
            pltpu.SemaphoreType.DMA,
            pltpu.SemaphoreType.DMA,
            pltpu.SemaphoreType.DMA,
            pltpu.SemaphoreType.DMA,
        ],
        compiler_params=pltpu.CompilerParams(use_tc_tiling_on_sc=False),
    )
    def k(idx_hbm, table_hbm, out_hbm, idx_v, rows0, rows1, sg0, sg1, so0, so1):
        wid = lax.axis_index("s") * info.num_cores + lax.axis_index("c")
        row0 = wid * n_chunks
        pltpu.sync_copy(idx_hbm.at[pl.ds(row0, n_chunks)], idx_v)

        def gather(g, rows, sem):
            return pltpu.async_copy(table_hbm.at[idx_v.at[g]], rows, sem)

        def out_at(g):
            return out_hbm.at[pl.ds((row0 + g) * chunk, chunk), pl.ds(0, D)]

        gather(0, rows0, sg0).wait()
        pltpu.async_copy(rows0, out_at(0), so0)
        gather(1, rows1, sg1).wait()
        pltpu.async_copy(rows1, out_at(1), so1)

        def body(i, carry):
            g = i * 2
            pltpu.make_async_copy(rows0, out_at(g - 2), so0).wait()
            gather(g, rows0, sg0).wait()
            pltpu.async_copy(rows0, out_at(g), so0)
            pltpu.make_async_copy(rows1, out_at(g - 1), so1).wait()
            gather(g + 1, rows1, sg1).wait()
            pltpu.async_copy(rows1, out_at(g + 1), so1)
            return carry

        lax.fori_loop(1, n_chunks // 2, body, 0)
        pltpu.make_async_copy(rows0, out_at(n_chunks - 2), so0).wait()
        pltpu.make_async_copy(rows1, out_at(n_chunks - 1), so1).wait()

    return k


def kernel(src_word, table):
    B = src_word.shape[0] * src_word.shape[1]
    D = table.shape[1]
    idx = src_word.reshape(B // _CHUNK, _CHUNK)
    outP = _make_gather(B, D, _CHUNK)(idx, table)
    outP = outP.reshape(src_word.shape + (2 * D,))
    return outP[:, :, :D]
